# Initial kernel scaffold; baseline (speedup 1.0000x reference)
import jax
import jax.numpy as jnp
from jax.experimental import pallas as pl


def kernel(x_RNA, x_ADT, sim_edge_index, sim_edge_weight, dist_edge_index, dist_edge_weight, common_edge_index, common_edge_weight, W_rna1, b_rna1, W_rna2, b_rna2, W_pro3, b_pro3, W_sim, b_sim, W_dist, b_dist, fusion1_W, fusion1_b, fusion2_W, fusion2_b):
    raise NotImplementedError("write your pallas kernel here")


# speedup vs baseline: 14.3539x; 14.3539x over previous
import functools

import jax
import jax.numpy as jnp
from jax import lax
from jax.experimental import pallas as pl
from jax.experimental.pallas import tpu as pltpu
from jax.experimental.pallas import tpu_sc as plsc

_N = 10000
_NPAD = 10240
_E = 320000
_W = 128
_NC, _NS = 2, 16
_NWORK = _NC * _NS
_EPW = 10240
_EP = _EPW * _NWORK
_NWIN = _EPW // _W
_B4 = 4
_NBATCH = _NWIN // _B4
_ROWS_PER_TILE = _N // _NS
_RCHUNK = 125
_DPT = _NPAD // _NS

_mesh = plsc.VectorSubcoreMesh(core_axis_name="c", subcore_axis_name="s")
_SC_PARAMS = pltpu.CompilerParams(needs_layout_passes=False,
                                  use_tc_tiling_on_sc=False)
_HIGH = lax.Precision.HIGHEST


def _zero_vec16():
    return jnp.zeros((16,), jnp.float32)


@functools.partial(
    pl.kernel,
    out_type=jax.ShapeDtypeStruct((_NC * 3 * _NPAD,), jnp.float32),
    mesh=_mesh,
    compiler_params=_SC_PARAMS,
    scratch_types=[
        pltpu.VMEM((_B4, _W), jnp.int32),
        pltpu.VMEM((_B4, _W), jnp.float32),
        pltpu.VMEM((_W,), jnp.float32),
        pltpu.VMEM_SHARED((_NPAD,), jnp.float32),
        pltpu.VMEM_SHARED((_NPAD,), jnp.float32),
        pltpu.VMEM_SHARED((_NPAD,), jnp.float32),
    ],
)
def _deg_kernel(col0, ew0, col1, ew1, col2, ew2, out,
                idx_v, val_v, zero_v, dacc0, dacc1, dacc2):
    c = lax.axis_index("c")
    s = lax.axis_index("s")
    wid = s * _NC + c
    graphs = [(col0, ew0, dacc0), (col1, ew1, dacc1), (col2, ew2, dacc2)]

    for j in range(_W // 16):
        zero_v[pl.ds(j * 16, 16)] = _zero_vec16()
    dbase = pl.multiple_of(s * _DPT, 128)
    for dacc in (dacc0, dacc1, dacc2):
        for k in range(_DPT // _W):
            pltpu.sync_copy(zero_v, dacc.at[pl.ds(dbase + k * _W, _W)])
    plsc.subcore_barrier()

    for colR, ewR, dacc in graphs:
        def body(b, _, colR=colR, ewR=ewR, dacc=dacc):
            r0 = wid * _NWIN + b * _B4
            pltpu.sync_copy(colR.at[pl.ds(r0, _B4)], idx_v)
            pltpu.sync_copy(ewR.at[pl.ds(r0, _B4)], val_v)
            for k in range(_B4):
                pltpu.sync_copy(val_v.at[k], dacc.at[idx_v.at[k]], add=True)
            return 0
        lax.fori_loop(0, _NBATCH, body, 0)

    plsc.subcore_barrier()
    for g, dacc in enumerate((dacc0, dacc1, dacc2)):
        obase = pl.multiple_of((c * 3 + g) * _NPAD + dbase, 128)
        pltpu.sync_copy(dacc.at[pl.ds(dbase, _DPT)],
                        out.at[pl.ds(obase, _DPT)])


def _make_agg(d):
    @functools.partial(
        pl.kernel,
        out_type=jax.ShapeDtypeStruct((_NC, _N, d), jnp.float32),
        mesh=_mesh,
        compiler_params=_SC_PARAMS,
        scratch_types=[
            pltpu.VMEM((_B4, _W), jnp.int32),
            pltpu.VMEM((_B4, _W), jnp.int32),
            pltpu.VMEM((_B4, _W), jnp.float32),
            pltpu.VMEM((_W,), jnp.float32),
            pltpu.VMEM((_W, d), jnp.float32),
            pltpu.VMEM((_NPAD,), jnp.float32),
            pltpu.VMEM_SHARED((_NPAD, d), jnp.float32),
            pltpu.SemaphoreType.DMA,
        ],
    )
    def agg(h, rowR, colR, ewR, dis, out,
            ri_v, ci_v, ew_v, norm_v, rows_v, dis_v, acc, sem):
        c = lax.axis_index("c")
        s = lax.axis_index("s")
        wid = s * _NC + c

        def zbody(r, _):
            for j in range(d // 16):
                rows_v[r, pl.ds(j * 16, 16)] = _zero_vec16()
            return 0
        lax.fori_loop(0, _W, zbody, 0)
        rbase = pl.multiple_of(s * _DPT, 128)
        for k in range(_DPT // _W):
            pltpu.sync_copy(rows_v,
                            acc.at[pl.ds(rbase + k * _W, _W)])
        pltpu.sync_copy(dis, dis_v)
        plsc.subcore_barrier()

        def body(b, _):
            r0 = wid * _NWIN + b * _B4
            pltpu.sync_copy(rowR.at[pl.ds(r0, _B4)], ri_v)
            pltpu.sync_copy(colR.at[pl.ds(r0, _B4)], ci_v)
            pltpu.sync_copy(ewR.at[pl.ds(r0, _B4)], ew_v)
            for k in range(_B4):
                pltpu.async_copy(h.at[ri_v.at[k]], rows_v, sem).wait()
                for i in range(_W // 16):
                    ridx = ri_v[k, pl.ds(i * 16, 16)]
                    cidx = ci_v[k, pl.ds(i * 16, 16)]
                    dr = plsc.load_gather(dis_v, [ridx])
                    dc = plsc.load_gather(dis_v, [cidx])
                    norm_v[pl.ds(i * 16, 16)] = (
                        dr * ew_v[k, pl.ds(i * 16, 16)] * dc)

                def sbody(g, _):
                    nv = norm_v[pl.ds(g * 16, 16)]
                    for e2 in range(16):
                        nrm = nv[e2]
                        e = g * 16 + e2
                        for j in range(d // 16):
                            rows_v[e, pl.ds(j * 16, 16)] = (
                                rows_v[e, pl.ds(j * 16, 16)] * nrm)
                    return 0
                lax.fori_loop(0, _W // 16, sbody, 0)
                pltpu.sync_copy(rows_v, acc.at[ci_v.at[k]], add=True)
            return 0
        lax.fori_loop(0, _NBATCH, body, 0)

        plsc.subcore_barrier()

        @pl.when(s < _NS - 1)
        def _drain_full():
            for k in range(_DPT // _W):
                r0 = rbase + k * _W
                pltpu.sync_copy(acc.at[pl.ds(r0, _W)],
                                out.at[c, pl.ds(r0, _W)])

        @pl.when(s == _NS - 1)
        def _drain_last():
            last = (_NS - 1) * _DPT
            for k in range(3):
                r0 = last + k * _W
                pltpu.sync_copy(acc.at[pl.ds(r0, _W)],
                                out.at[c, pl.ds(r0, _W)])
            r0 = last + 3 * _W
            pltpu.sync_copy(acc.at[pl.ds(r0, _N - r0)],
                            out.at[c, pl.ds(r0, _N - r0)])
    return agg


_agg128 = _make_agg(128)
_agg64 = _make_agg(64)
_agg16 = _make_agg(16)


def _tc_pre(x, w1, w2, degp):
    def body(x_ref, w1_ref, w2_ref, deg_ref, h1_ref, h2_ref, dis_ref):
        xb = x_ref[...]
        h1_ref[...] = lax.dot(xb, w1_ref[...], precision=_HIGH,
                              preferred_element_type=jnp.float32)
        h2_ref[...] = lax.dot(xb, w2_ref[...], precision=_HIGH,
                              preferred_element_type=jnp.float32)
        deg = deg_ref[0] + deg_ref[1] + 1.0
        dis_ref[...] = jnp.where(deg > 0, 1.0 / jnp.sqrt(deg), 0.0)

    return pl.pallas_call(
        body,
        grid=(10,),
        in_specs=[
            pl.BlockSpec((1000, 128), lambda i: (i, 0)),
            pl.BlockSpec((128, 128), lambda i: (0, 0)),
            pl.BlockSpec((128, 128), lambda i: (0, 0)),
            pl.BlockSpec((2, 3, 1024), lambda i: (0, 0, i)),
        ],
        out_specs=[
            pl.BlockSpec((1000, 128), lambda i: (i, 0)),
            pl.BlockSpec((1000, 128), lambda i: (i, 0)),
            pl.BlockSpec((3, 1024), lambda i: (0, i)),
        ],
        out_shape=[
            jax.ShapeDtypeStruct((_N, 128), jnp.float32),
            jax.ShapeDtypeStruct((_N, 128), jnp.float32),
            jax.ShapeDtypeStruct((3, _NPAD), jnp.float32),
        ],
    )(x, w1, w2, degp)


def _tc_mid(a1, a2, a5, h1, h2, xadt, d0, d1, d2,
            b1, b2, w_sim, w_dist, w_pro, b_pro):
    def body(a1_r, a2_r, a5_r, h1_r, h2_r, xadt_r, d0_r, d1_r, d2_r,
             b1_r, b2_r, ws_r, wd_r, wp_r, bp_r, h3_o, h4_o, pro_o):
        d0b = d0_r[...]
        d1b = d1_r[...]
        d2b = d2_r[...]
        out1 = a1_r[0] + a1_r[1] + d0b * d0b * h1_r[...] + b1_r[...]
        xs = jnp.maximum(out1, 0.0)
        h3_o[...] = lax.dot(xs, ws_r[...], precision=_HIGH,
                            preferred_element_type=jnp.float32)
        out2 = a2_r[0] + a2_r[1] + d1b * d1b * h2_r[...] + b2_r[...]
        xd = jnp.maximum(out2, 0.0)
        h4_o[...] = lax.dot(xd, wd_r[...], precision=_HIGH,
                            preferred_element_type=jnp.float32)
        agg5 = a5_r[0] + a5_r[1] + d2b * d2b * xadt_r[...]
        pro_o[...] = lax.dot(agg5, wp_r[...], precision=_HIGH,
                             preferred_element_type=jnp.float32) + bp_r[...]

    return pl.pallas_call(
        body,
        grid=(10,),
        in_specs=[
            pl.BlockSpec((2, 1000, 128), lambda i: (0, i, 0)),
            pl.BlockSpec((2, 1000, 128), lambda i: (0, i, 0)),
            pl.BlockSpec((2, 1000, 16), lambda i: (0, i, 0)),
            pl.BlockSpec((1000, 128), lambda i: (i, 0)),
            pl.BlockSpec((1000, 128), lambda i: (i, 0)),
            pl.BlockSpec((1000, 16), lambda i: (i, 0)),
            pl.BlockSpec((1000, 1), lambda i: (i, 0)),
            pl.BlockSpec((1000, 1), lambda i: (i, 0)),
            pl.BlockSpec((1000, 1), lambda i: (i, 0)),
            pl.BlockSpec((1, 128), lambda i: (0, 0)),
            pl.BlockSpec((1, 128), lambda i: (0, 0)),
            pl.BlockSpec((128, 64), lambda i: (0, 0)),
            pl.BlockSpec((128, 64), lambda i: (0, 0)),
            pl.BlockSpec((16, 64), lambda i: (0, 0)),
            pl.BlockSpec((1, 64), lambda i: (0, 0)),
        ],
        out_specs=[
            pl.BlockSpec((1000, 64), lambda i: (i, 0)),
            pl.BlockSpec((1000, 64), lambda i: (i, 0)),
            pl.BlockSpec((1000, 64), lambda i: (i, 0)),
        ],
        out_shape=[
            jax.ShapeDtypeStruct((_N, 64), jnp.float32),
            jax.ShapeDtypeStruct((_N, 64), jnp.float32),
            jax.ShapeDtypeStruct((_N, 64), jnp.float32),
        ],
    )(a1, a2, a5, h1, h2, xadt, d0, d1, d2, b1, b2,
      w_sim, w_dist, w_pro, b_pro)


def _tc_post(a3, a4, h3, h4, d0, d1, bs, bd, f1w, f1b, f2w, f2b, pro):
    def body(a3_r, a4_r, h3_r, h4_r, d0_r, d1_r, bs_r, bd_r,
             f1w_r, f1b_r, f2w_r, f2b_r, pro_r,
             xsim_o, xdist_o, fused_o, fp_o):
        d0b = d0_r[...]
        d1b = d1_r[...]
        xsim = a3_r[0] + a3_r[1] + d0b * d0b * h3_r[...] + bs_r[...]
        xdist = a4_r[0] + a4_r[1] + d1b * d1b * h4_r[...] + bd_r[...]
        xsim_o[...] = xsim
        xdist_o[...] = xdist
        fused = (lax.dot(xsim, f1w_r[0:64], precision=_HIGH,
                         preferred_element_type=jnp.float32)
                 + lax.dot(xdist, f1w_r[64:128], precision=_HIGH,
                           preferred_element_type=jnp.float32)
                 + f1b_r[...])
        fused_o[...] = fused
        fp_o[...] = (lax.dot(fused, f2w_r[0:64], precision=_HIGH,
                             preferred_element_type=jnp.float32)
                     + lax.dot(pro_r[...], f2w_r[64:128], precision=_HIGH,
                               preferred_element_type=jnp.float32)
                     + f2b_r[...])

    return pl.pallas_call(
        body,
        grid=(10,),
        in_specs=[
            pl.BlockSpec((2, 1000, 64), lambda i: (0, i, 0)),
            pl.BlockSpec((2, 1000, 64), lambda i: (0, i, 0)),
            pl.BlockSpec((1000, 64), lambda i: (i, 0)),
            pl.BlockSpec((1000, 64), lambda i: (i, 0)),
            pl.BlockSpec((1000, 1), lambda i: (i, 0)),
            pl.BlockSpec((1000, 1), lambda i: (i, 0)),
            pl.BlockSpec((1, 64), lambda i: (0, 0)),
            pl.BlockSpec((1, 64), lambda i: (0, 0)),
            pl.BlockSpec((128, 64), lambda i: (0, 0)),
            pl.BlockSpec((1, 64), lambda i: (0, 0)),
            pl.BlockSpec((128, 64), lambda i: (0, 0)),
            pl.BlockSpec((1, 64), lambda i: (0, 0)),
            pl.BlockSpec((1000, 64), lambda i: (i, 0)),
        ],
        out_specs=[
            pl.BlockSpec((1000, 64), lambda i: (i, 0)),
            pl.BlockSpec((1000, 64), lambda i: (i, 0)),
            pl.BlockSpec((1000, 64), lambda i: (i, 0)),
            pl.BlockSpec((1000, 64), lambda i: (i, 0)),
        ],
        out_shape=[
            jax.ShapeDtypeStruct((_N, 64), jnp.float32),
            jax.ShapeDtypeStruct((_N, 64), jnp.float32),
            jax.ShapeDtypeStruct((_N, 64), jnp.float32),
            jax.ShapeDtypeStruct((_N, 64), jnp.float32),
        ],
    )(a3, a4, h3, h4, d0, d1, bs, bd, f1w, f1b, f2w, f2b, pro)


def kernel(x_RNA, x_ADT, sim_edge_index, sim_edge_weight,
           dist_edge_index, dist_edge_weight,
           common_edge_index, common_edge_weight,
           W_rna1, b_rna1, W_rna2, b_rna2, W_pro3, b_pro3,
           W_sim, b_sim, W_dist, b_dist,
           fusion1_W, fusion1_b, fusion2_W, fusion2_b):
    f32 = jnp.float32
    pad = _EP - _E
    pad_idx = lax.iota(jnp.int32, pad) % _N

    def prep(ei, ew):
        row = jnp.concatenate([ei[0], pad_idx]).reshape(_EP // _W, _W)
        col = jnp.concatenate([ei[1], pad_idx]).reshape(_EP // _W, _W)
        w = jnp.concatenate([ew, jnp.zeros((pad,), f32)]).reshape(
            _EP // _W, _W)
        return row, col, w

    rs, cs, ws_ = prep(sim_edge_index, sim_edge_weight)
    rd, cd, wd_ = prep(dist_edge_index, dist_edge_weight)
    rc, cc, wc_ = prep(common_edge_index, common_edge_weight)

    degp = _deg_kernel(cs, ws_, cd, wd_, cc, wc_).reshape(_NC, 3, _NPAD)
    h1, h2, dis3 = _tc_pre(x_RNA, W_rna1, W_rna2, degp)
    dis_s, dis_d, dis_c = dis3[0], dis3[1], dis3[2]

    a1 = _agg128(h1, rs, cs, ws_, dis_s)
    a2 = _agg128(h2, rd, cd, wd_, dis_d)
    a5 = _agg16(x_ADT, rc, cc, wc_, dis_c)

    d0 = dis_s[:_N, None]
    d1 = dis_d[:_N, None]
    d2 = dis_c[:_N, None]
    h3, h4, pro = _tc_mid(a1, a2, a5, h1, h2, x_ADT, d0, d1, d2,
                          b_rna1[None], b_rna2[None],
                          W_sim, W_dist, W_pro3, b_pro3[None])

    a3 = _agg64(h3, rs, cs, ws_, dis_s)
    a4 = _agg64(h4, rd, cd, wd_, dis_d)

    x_sim, x_dist, fused, fused_pro = _tc_post(
        a3, a4, h3, h4, d0, d1, b_sim[None], b_dist[None],
        fusion1_W, fusion1_b[None], fusion2_W, fusion2_b[None], pro)
    return (x_sim, x_dist, fused, fused_pro, pro)
